# Initial kernel scaffold; baseline (speedup 1.0000x reference)
import jax
import jax.numpy as jnp
from jax.experimental import pallas as pl


def kernel(x, edge_index, edge_weight, W_gcn, b_gcn, bn_w, bn_b, bn_mean, bn_var, W_ih0, W_hh0, b_ih0, b_hh0, W_ih1, W_hh1, b_ih1, b_hh1, out_W1, out_b1, out_W2, out_b2):
    raise NotImplementedError("write your pallas kernel here")


# speedup vs baseline: 16.0673x; 16.0673x over previous
import functools

import jax
import jax.numpy as jnp
from jax import lax
from jax.experimental import pallas as pl
from jax.experimental.pallas import tpu as pltpu
from jax.experimental.pallas import tpu_sc as plsc

N = 10000
E = 160000
D_IN = 256
D_GCN = 256
D_GRU = 128
D_GAT = 64
HALF = 128

NSUB = 16
ET = E // NSUB
CH = 80
NCHUNK = ET // CH
RB = 40
NT_BIG = 640
NT_LAST = 400
NPAD = 10240
DT = NPAD // NSUB


def _rsqrt_newton(v):
    i = lax.bitcast_convert_type(v, jnp.int32)
    i = jnp.int32(0x5F3759DF) - lax.shift_right_logical(i, 1)
    y = lax.bitcast_convert_type(i, jnp.float32)
    for _ in range(3):
        y = y * (1.5 - 0.5 * v * y * y)
    return y


NB = 4
NG = 124 // NB

CH2 = 96
NCH2 = ET // CH2
TAIL_E = ET - NCH2 * CH2
NB2 = 4
NG2 = NCH2 // NB2


def _sc_degree(col_ids, edge_weight):
    mesh = plsc.VectorSubcoreMesh(core_axis_name="c", subcore_axis_name="s")

    @functools.partial(
        pl.kernel,
        out_type=jax.ShapeDtypeStruct((NPAD,), jnp.float32),
        mesh=mesh,
        compiler_params=pltpu.CompilerParams(needs_layout_passes=False),
        scratch_types=dict(
            col2d=pltpu.VMEM((NCHUNK, CH), jnp.int32),
            ew2d=pltpu.VMEM((NCHUNK, CH), jnp.float32),
            dbuf=pltpu.VMEM((DT,), jnp.float32),
            deg_s=pltpu.VMEM_SHARED((NPAD,), jnp.float32),
            stage_sem=pltpu.SemaphoreType.DMA,
            deg_sem=pltpu.SemaphoreType.DMA,
        ),
    )
    def deg_kernel(cid_hbm, ew_hbm, out_hbm, *, col2d, ew2d, dbuf, deg_s,
                   stage_sem, deg_sem):
        core = lax.axis_index("c")
        sub = lax.axis_index("s")

        @pl.when(core == 0)
        def _():
            ebase = sub * ET
            def stage_body(i, _):
                base = ebase + i * CH
                pltpu.async_copy(cid_hbm.at[pl.ds(base, CH)], col2d.at[i],
                                 stage_sem)
                pltpu.async_copy(ew_hbm.at[pl.ds(base, CH)], ew2d.at[i],
                                 stage_sem)
                return 0
            lax.fori_loop(0, NCHUNK, stage_body, 0)
            zero16 = jnp.zeros((16,), jnp.float32)
            def z_body(j, _):
                dbuf[pl.ds(j * 16, 16)] = zero16
                return 0
            lax.fori_loop(0, DT // 16, z_body, 0)
            pltpu.sync_copy(dbuf, deg_s.at[pl.ds(sub * DT, DT)])
            def stage_drain(i, _):
                pltpu.make_async_copy(cid_hbm.at[pl.ds(ebase, CH)],
                                      col2d.at[0], stage_sem).wait()
                pltpu.make_async_copy(ew_hbm.at[pl.ds(ebase, CH)],
                                      ew2d.at[0], stage_sem).wait()
                return 0
            lax.fori_loop(0, NCHUNK, stage_drain, 0)
            plsc.subcore_barrier()

            def deg_body(i, _):
                pltpu.async_copy(ew2d.at[i], deg_s.at[col2d.at[i]], deg_sem,
                                 add=True)
                return 0
            lax.fori_loop(0, NCHUNK, deg_body, 0)
            def deg_drain(i, _):
                pltpu.make_async_copy(ew2d.at[0], deg_s.at[col2d.at[0]],
                                      deg_sem).wait()
                return 0
            lax.fori_loop(0, NCHUNK, deg_drain, 0)
            plsc.subcore_barrier()

            pltpu.sync_copy(deg_s.at[pl.ds(sub * DT, DT)], dbuf)
            def dinv_body(j, _):
                v = dbuf[pl.ds(j * 16, 16)] + 1.0
                dbuf[pl.ds(j * 16, 16)] = _rsqrt_newton(v)
                return 0
            lax.fori_loop(0, DT // 16, dinv_body, 0)
            pltpu.sync_copy(dbuf, out_hbm.at[pl.ds(sub * DT, DT)])

    return deg_kernel(col_ids, edge_weight)


def _sc_aggregate(y_flat, row_ids, col_ids, edge_weight):
    mesh = plsc.VectorSubcoreMesh(core_axis_name="c", subcore_axis_name="s")

    @functools.partial(
        pl.kernel,
        out_type=jax.ShapeDtypeStruct((2 * N, HALF), jnp.float32),
        mesh=mesh,
        compiler_params=pltpu.CompilerParams(needs_layout_passes=False),
        scratch_types=dict(
            row_b=[pltpu.VMEM((CH2,), jnp.int32) for _ in range(NB2)],
            col_b=[pltpu.VMEM((CH2,), jnp.int32) for _ in range(NB2)],
            ew_b=[pltpu.VMEM((CH2,), jnp.float32) for _ in range(NB2)],
            col_t=pltpu.VMEM((TAIL_E,), jnp.int32),
            msg=[pltpu.VMEM((CH2, HALF), jnp.float32) for _ in range(NB2)],
            isem=pltpu.SemaphoreType.DMA,
            stsem=[pltpu.SemaphoreType.DMA for _ in range(NB2)],
            gsem=[pltpu.SemaphoreType.DMA for _ in range(NB2)],
            ssem=[pltpu.SemaphoreType.DMA for _ in range(NB2)],
            acc=pltpu.VMEM_SHARED((N, HALF), jnp.float32),
        ),
    )
    def agg_kernel(y_hbm, rid_hbm, cid_hbm, ew_hbm, out_hbm, *,
                   row_b, col_b, ew_b, col_t, msg, isem, stsem, gsem,
                   ssem, acc):
        core = lax.axis_index("c")
        sub = lax.axis_index("s")
        ebase = sub * ET
        coreofs = core * N
        nbase = sub * NT_BIG
        last = sub == NSUB - 1

        @pl.when(~last)
        def _():
            pltpu.async_copy(y_hbm.at[pl.ds(coreofs + nbase, NT_BIG)],
                             acc.at[pl.ds(nbase, NT_BIG)], isem)
        @pl.when(last)
        def _():
            pltpu.async_copy(y_hbm.at[pl.ds(coreofs + nbase, NT_LAST)],
                             acc.at[pl.ds(nbase, NT_LAST)], isem)

        def fire_stage(c, j):
            base = ebase + c * CH2
            pltpu.async_copy(rid_hbm.at[pl.ds(base, CH2)], row_b[j], stsem[j])
            pltpu.async_copy(cid_hbm.at[pl.ds(base, CH2)], col_b[j], stsem[j])
            pltpu.async_copy(ew_hbm.at[pl.ds(base, CH2)], ew_b[j], stsem[j])
        def wait_stage(j):
            pltpu.make_async_copy(rid_hbm.at[pl.ds(ebase, CH2)], row_b[j],
                                  stsem[j]).wait()
            pltpu.make_async_copy(cid_hbm.at[pl.ds(ebase, CH2)], col_b[j],
                                  stsem[j]).wait()
            pltpu.make_async_copy(ew_hbm.at[pl.ds(ebase, CH2)], ew_b[j],
                                  stsem[j]).wait()
        def fire_gather(j):
            pltpu.async_copy(y_hbm.at[row_b[j]], msg[j], gsem[j])
        def wait_gather(j):
            pltpu.make_async_copy(y_hbm.at[row_b[j]], msg[j], gsem[j]).wait()
        def fire_scatter(j):
            pltpu.async_copy(msg[j], acc.at[col_b[j]], ssem[j], add=True)
        def wait_scatter(j):
            pltpu.make_async_copy(msg[j], acc.at[col_b[j]], ssem[j]).wait()
        def to_gather_ids(j):
            for q in range(CH2 // 16):
                row_b[j][pl.ds(q * 16, 16)] = (
                    row_b[j][pl.ds(q * 16, 16)] + coreofs)
        def scale_msg(j):
            def mul_body(e, _):
                s = plsc.load_gather(ew_b[j], [jnp.full((16,), e, jnp.int32)])
                for v in range(HALF // 16):
                    msg[j][e, pl.ds(v * 16, 16)] = (
                        msg[j][e, pl.ds(v * 16, 16)] * s)
                return 0
            lax.fori_loop(0, CH2, mul_body, 0, unroll=8)

        for j in range(NB2):
            fire_stage(j, j)

        @pl.when(~last)
        def _():
            pltpu.make_async_copy(y_hbm.at[pl.ds(coreofs, NT_BIG)],
                                  acc.at[pl.ds(0, NT_BIG)], isem).wait()
        @pl.when(last)
        def _():
            pltpu.make_async_copy(y_hbm.at[pl.ds(coreofs, NT_LAST)],
                                  acc.at[pl.ds(0, NT_LAST)], isem).wait()
        plsc.subcore_barrier()

        def group(g, _):
            c0 = g * NB2
            for j in range(NB2):
                wait_stage(j)
                to_gather_ids(j)
                fire_gather(j)
            for j in range(NB2):
                wait_gather(j)
                scale_msg(j)
                fire_scatter(j)
            for j in range(NB2):
                cn = c0 + NB2 + j

                @pl.when(cn < NCH2)
                def _():
                    wait_scatter(j)
                    fire_stage(cn, j)
            return 0
        lax.fori_loop(0, NG2, group, 0)
        tbase = ebase + NCH2 * CH2
        wait_scatter(0)
        pltpu.sync_copy(rid_hbm.at[pl.ds(tbase, TAIL_E)],
                        row_b[0].at[pl.ds(0, TAIL_E)])
        pltpu.sync_copy(cid_hbm.at[pl.ds(tbase, TAIL_E)], col_t)
        pltpu.sync_copy(ew_hbm.at[pl.ds(tbase, TAIL_E)],
                        ew_b[0].at[pl.ds(0, TAIL_E)])
        row_b[0][pl.ds(0, 16)] = row_b[0][pl.ds(0, 16)] + coreofs
        pltpu.sync_copy(y_hbm.at[row_b[0].at[pl.ds(0, TAIL_E)]],
                        msg[0].at[pl.ds(0, TAIL_E)])
        def tmul_body(e, _):
            s = plsc.load_gather(ew_b[0], [jnp.full((16,), e, jnp.int32)])
            for v in range(HALF // 16):
                msg[0][e, pl.ds(v * 16, 16)] = msg[0][e, pl.ds(v * 16, 16)] * s
            return 0
        lax.fori_loop(0, TAIL_E, tmul_body, 0, unroll=8)
        pltpu.sync_copy(msg[0].at[pl.ds(0, TAIL_E)], acc.at[col_t], add=True)
        for j in range(1, NB2):
            wait_scatter(j)
        plsc.subcore_barrier()

        @pl.when(~last)
        def _():
            pltpu.sync_copy(acc.at[pl.ds(nbase, NT_BIG)],
                            out_hbm.at[pl.ds(coreofs + nbase, NT_BIG)])
        @pl.when(last)
        def _():
            pltpu.sync_copy(acc.at[pl.ds(nbase, NT_LAST)],
                            out_hbm.at[pl.ds(coreofs + nbase, NT_LAST)])

    return agg_kernel(y_flat, row_ids, col_ids, edge_weight)


BS = 1000


def _tc_matmul_kernel(x_ref, w_ref, dinv_ref, o_ref):
    res = lax.dot_general(x_ref[...], w_ref[...],
                          (((1,), (1,)), ((), ())),
                          preferred_element_type=jnp.float32)
    res = res * dinv_ref[...]
    o_ref[0] = res[:, :HALF]
    o_ref[1] = res[:, HALF:]


def _tc_head_kernel(a_ref, dinv_ref, sc0_ref, sc1_ref, sh0_ref, sh1_ref,
                    wih0a_ref, wih0b_ref, bi0_ref, bh0_ref,
                    wih1_ref, bi1_ref, bh1_ref,
                    w1_ref, b1_ref, w2_ref, b2_ref, o_ref):
    dv = dinv_ref[...]
    s0 = jnp.maximum((a_ref[0] * dv) * sc0_ref[...] + sh0_ref[...], 0.0)
    s1 = jnp.maximum((a_ref[1] * dv) * sc1_ref[...] + sh1_ref[...], 0.0)
    gi = (lax.dot_general(s0, wih0a_ref[...], (((1,), (1,)), ((), ())),
                          preferred_element_type=jnp.float32)
          + lax.dot_general(s1, wih0b_ref[...], (((1,), (1,)), ((), ())),
                            preferred_element_type=jnp.float32)
          + bi0_ref[...])
    bh0 = bh0_ref[...]
    H = D_GRU
    r = jax.nn.sigmoid(gi[:, :H] + bh0[:, :H])
    z = jax.nn.sigmoid(gi[:, H:2 * H] + bh0[:, H:2 * H])
    n_ = jnp.tanh(gi[:, 2 * H:] + r * bh0[:, 2 * H:])
    h1 = (1.0 - z) * n_
    gi2 = lax.dot_general(h1, wih1_ref[...], (((1,), (1,)), ((), ())),
                          preferred_element_type=jnp.float32) + bi1_ref[...]
    bh1 = bh1_ref[...]
    r2 = jax.nn.sigmoid(gi2[:, :H] + bh1[:, :H])
    z2 = jax.nn.sigmoid(gi2[:, H:2 * H] + bh1[:, H:2 * H])
    n2 = jnp.tanh(gi2[:, 2 * H:] + r2 * bh1[:, 2 * H:])
    h2 = (1.0 - z2) * n2
    hid = jnp.maximum(
        lax.dot_general(h2, w1_ref[...], (((1,), (1,)), ((), ())),
                        preferred_element_type=jnp.float32) + b1_ref[...], 0.0)
    o_ref[...] = (jnp.sum(hid * w2_ref[...], axis=1, keepdims=True)
                  + b2_ref[...])


def _full(shape):
    return pl.BlockSpec(shape, lambda i: tuple(0 for _ in shape))


def kernel(x, edge_index, edge_weight, W_gcn, b_gcn, bn_w, bn_b, bn_mean,
           bn_var, W_ih0, W_hh0, b_ih0, b_hh0, W_ih1, W_hh1, b_ih1, b_hh1,
           out_W1, out_b1, out_W2, out_b2):
    dinv_pad = _sc_degree(edge_index[1], edge_weight)
    dinv_col = dinv_pad[:N].reshape(N, 1)

    y2 = pl.pallas_call(
        _tc_matmul_kernel,
        grid=(N // BS,),
        in_specs=[pl.BlockSpec((BS, D_IN), lambda i: (i, 0)),
                  _full((D_GCN, D_IN)),
                  pl.BlockSpec((BS, 1), lambda i: (i, 0))],
        out_specs=pl.BlockSpec((2, BS, HALF), lambda i: (0, i, 0)),
        out_shape=jax.ShapeDtypeStruct((2, N, HALF), jnp.float32),
    )(x, W_gcn, dinv_col)
    y_flat = y2.reshape(2 * N, HALF)

    agg_flat = _sc_aggregate(y_flat, edge_index[0], edge_index[1],
                             edge_weight)
    agg = agg_flat.reshape(2, N, HALF)

    bscale = bn_w * lax.rsqrt(bn_var + 1e-5)
    bshift = (b_gcn - bn_mean) * bscale + bn_b
    score = pl.pallas_call(
        _tc_head_kernel,
        grid=(N // BS,),
        in_specs=[pl.BlockSpec((2, BS, HALF), lambda i: (0, i, 0)),
                  pl.BlockSpec((BS, 1), lambda i: (i, 0)),
                  _full((1, HALF)), _full((1, HALF)),
                  _full((1, HALF)), _full((1, HALF)),
                  _full((3 * D_GRU, HALF)), _full((3 * D_GRU, HALF)),
                  _full((1, 3 * D_GRU)), _full((1, 3 * D_GRU)),
                  _full((3 * D_GRU, D_GRU)),
                  _full((1, 3 * D_GRU)), _full((1, 3 * D_GRU)),
                  _full((D_GAT, D_GRU)), _full((1, D_GAT)),
                  _full((1, D_GAT)), _full((1, 1))],
        out_specs=pl.BlockSpec((BS, 1), lambda i: (i, 0)),
        out_shape=jax.ShapeDtypeStruct((N, 1), jnp.float32),
    )(agg, dinv_col,
      bscale[:HALF].reshape(1, HALF), bscale[HALF:].reshape(1, HALF),
      bshift[:HALF].reshape(1, HALF), bshift[HALF:].reshape(1, HALF),
      W_ih0[:, :HALF], W_ih0[:, HALF:],
      b_ih0.reshape(1, -1), b_hh0.reshape(1, -1),
      W_ih1, b_ih1.reshape(1, -1), b_hh1.reshape(1, -1),
      out_W1, out_b1.reshape(1, -1),
      out_W2, out_b2.reshape(1, -1))
    return score


# speedup vs baseline: 18.0091x; 1.1209x over previous
import functools

import jax
import jax.numpy as jnp
from jax import lax
from jax.experimental import pallas as pl
from jax.experimental.pallas import tpu as pltpu
from jax.experimental.pallas import tpu_sc as plsc

N = 10000
E = 160000
D_IN = 256
D_GCN = 256
D_GRU = 128
D_GAT = 64
HALF = 128

NSUB = 16
ET = E // NSUB
CH = 80
NCHUNK = ET // CH
RB = 40
NT_BIG = 640
NT_LAST = 400
NPAD = 10240
DT = NPAD // NSUB


def _rsqrt_newton(v):
    i = lax.bitcast_convert_type(v, jnp.int32)
    i = jnp.int32(0x5F3759DF) - lax.shift_right_logical(i, 1)
    y = lax.bitcast_convert_type(i, jnp.float32)
    for _ in range(3):
        y = y * (1.5 - 0.5 * v * y * y)
    return y


NB = 4
NG = 124 // NB

CH2 = 96
NCH2 = ET // CH2
TAIL_E = ET - NCH2 * CH2
NB2 = 4
NG2 = NCH2 // NB2


def _sc_degree(col_ids, edge_weight):
    mesh = plsc.VectorSubcoreMesh(core_axis_name="c", subcore_axis_name="s")

    @functools.partial(
        pl.kernel,
        out_type=jax.ShapeDtypeStruct((NPAD,), jnp.float32),
        mesh=mesh,
        compiler_params=pltpu.CompilerParams(needs_layout_passes=False),
        scratch_types=dict(
            col2d=pltpu.VMEM((NCHUNK, CH), jnp.int32),
            ew2d=pltpu.VMEM((NCHUNK, CH), jnp.float32),
            dbuf=pltpu.VMEM((DT,), jnp.float32),
            deg_s=pltpu.VMEM_SHARED((NPAD,), jnp.float32),
            stage_sem=pltpu.SemaphoreType.DMA,
            deg_sem=pltpu.SemaphoreType.DMA,
        ),
    )
    def deg_kernel(cid_hbm, ew_hbm, out_hbm, *, col2d, ew2d, dbuf, deg_s,
                   stage_sem, deg_sem):
        core = lax.axis_index("c")
        sub = lax.axis_index("s")

        @pl.when(core == 0)
        def _():
            ebase = sub * ET
            def stage_body(i, _):
                base = ebase + i * CH
                pltpu.async_copy(cid_hbm.at[pl.ds(base, CH)], col2d.at[i],
                                 stage_sem)
                pltpu.async_copy(ew_hbm.at[pl.ds(base, CH)], ew2d.at[i],
                                 stage_sem)
                return 0
            lax.fori_loop(0, NCHUNK, stage_body, 0)
            zero16 = jnp.zeros((16,), jnp.float32)
            def z_body(j, _):
                dbuf[pl.ds(j * 16, 16)] = zero16
                return 0
            lax.fori_loop(0, DT // 16, z_body, 0)
            pltpu.sync_copy(dbuf, deg_s.at[pl.ds(sub * DT, DT)])
            def stage_drain(i, _):
                pltpu.make_async_copy(cid_hbm.at[pl.ds(ebase, CH)],
                                      col2d.at[0], stage_sem).wait()
                pltpu.make_async_copy(ew_hbm.at[pl.ds(ebase, CH)],
                                      ew2d.at[0], stage_sem).wait()
                return 0
            lax.fori_loop(0, NCHUNK, stage_drain, 0)
            plsc.subcore_barrier()

            def deg_body(i, _):
                pltpu.async_copy(ew2d.at[i], deg_s.at[col2d.at[i]], deg_sem,
                                 add=True)
                return 0
            lax.fori_loop(0, NCHUNK, deg_body, 0)
            def deg_drain(i, _):
                pltpu.make_async_copy(ew2d.at[0], deg_s.at[col2d.at[0]],
                                      deg_sem).wait()
                return 0
            lax.fori_loop(0, NCHUNK, deg_drain, 0)
            plsc.subcore_barrier()

            pltpu.sync_copy(deg_s.at[pl.ds(sub * DT, DT)], dbuf)
            def dinv_body(j, _):
                v = dbuf[pl.ds(j * 16, 16)] + 1.0
                dbuf[pl.ds(j * 16, 16)] = _rsqrt_newton(v)
                return 0
            lax.fori_loop(0, DT // 16, dinv_body, 0)
            pltpu.sync_copy(dbuf, out_hbm.at[pl.ds(sub * DT, DT)])

    return deg_kernel(col_ids, edge_weight)


def _sc_aggregate(y_flat, row_ids, col_ids, edge_weight):
    mesh = plsc.VectorSubcoreMesh(core_axis_name="c", subcore_axis_name="s")

    @functools.partial(
        pl.kernel,
        out_type=jax.ShapeDtypeStruct((2 * N, HALF), jnp.float32),
        mesh=mesh,
        compiler_params=pltpu.CompilerParams(needs_layout_passes=False),
        scratch_types=dict(
            row_b=[pltpu.VMEM((CH2,), jnp.int32) for _ in range(NB2)],
            col_b=[pltpu.VMEM((CH2,), jnp.int32) for _ in range(NB2)],
            ew_b=[pltpu.VMEM((CH2,), jnp.float32) for _ in range(NB2)],
            col_t=pltpu.VMEM((TAIL_E,), jnp.int32),
            msg=[pltpu.VMEM((CH2, HALF), jnp.float32) for _ in range(NB2)],
            isem=pltpu.SemaphoreType.DMA,
            stsem=[pltpu.SemaphoreType.DMA for _ in range(NB2)],
            gsem=[pltpu.SemaphoreType.DMA for _ in range(NB2)],
            ssem=[pltpu.SemaphoreType.DMA for _ in range(NB2)],
            acc=pltpu.VMEM_SHARED((N, HALF), jnp.float32),
        ),
    )
    def agg_kernel(y_hbm, rid_hbm, cid_hbm, ew_hbm, out_hbm, *,
                   row_b, col_b, ew_b, col_t, msg, isem, stsem, gsem,
                   ssem, acc):
        core = lax.axis_index("c")
        sub = lax.axis_index("s")
        ebase = sub * ET
        coreofs = core * N
        nbase = sub * NT_BIG
        last = sub == NSUB - 1

        @pl.when(~last)
        def _():
            pltpu.async_copy(y_hbm.at[pl.ds(coreofs + nbase, NT_BIG)],
                             acc.at[pl.ds(nbase, NT_BIG)], isem)
        @pl.when(last)
        def _():
            pltpu.async_copy(y_hbm.at[pl.ds(coreofs + nbase, NT_LAST)],
                             acc.at[pl.ds(nbase, NT_LAST)], isem)

        def fire_stage(c, j):
            base = ebase + c * CH2
            pltpu.async_copy(rid_hbm.at[pl.ds(base, CH2)], row_b[j], stsem[j])
            pltpu.async_copy(cid_hbm.at[pl.ds(base, CH2)], col_b[j], stsem[j])
            pltpu.async_copy(ew_hbm.at[pl.ds(base, CH2)], ew_b[j], stsem[j])
        def wait_stage(j):
            pltpu.make_async_copy(rid_hbm.at[pl.ds(ebase, CH2)], row_b[j],
                                  stsem[j]).wait()
            pltpu.make_async_copy(cid_hbm.at[pl.ds(ebase, CH2)], col_b[j],
                                  stsem[j]).wait()
            pltpu.make_async_copy(ew_hbm.at[pl.ds(ebase, CH2)], ew_b[j],
                                  stsem[j]).wait()
        def fire_gather(j):
            pltpu.async_copy(y_hbm.at[row_b[j]], msg[j], gsem[j])
        def wait_gather(j):
            pltpu.make_async_copy(y_hbm.at[row_b[j]], msg[j], gsem[j]).wait()
        def fire_scatter(j):
            pltpu.async_copy(msg[j], acc.at[col_b[j]], ssem[j], add=True)
        def wait_scatter(j):
            pltpu.make_async_copy(msg[j], acc.at[col_b[j]], ssem[j]).wait()
        def to_gather_ids(j):
            for q in range(CH2 // 16):
                row_b[j][pl.ds(q * 16, 16)] = (
                    row_b[j][pl.ds(q * 16, 16)] + coreofs)
        def scale_msg(j):
            @plsc.parallel_loop(0, CH2, unroll=8)
            def _(e):
                s = plsc.load_gather(ew_b[j], [jnp.full((16,), e, jnp.int32)])
                for v in range(HALF // 16):
                    msg[j][e, pl.ds(v * 16, 16)] = (
                        msg[j][e, pl.ds(v * 16, 16)] * s)

        for j in range(NB2):
            fire_stage(j, j)

        @pl.when(~last)
        def _():
            pltpu.make_async_copy(y_hbm.at[pl.ds(coreofs, NT_BIG)],
                                  acc.at[pl.ds(0, NT_BIG)], isem).wait()
        @pl.when(last)
        def _():
            pltpu.make_async_copy(y_hbm.at[pl.ds(coreofs, NT_LAST)],
                                  acc.at[pl.ds(0, NT_LAST)], isem).wait()
        plsc.subcore_barrier()

        def group(g, _):
            c0 = g * NB2
            for j in range(NB2):
                wait_stage(j)
                to_gather_ids(j)
                fire_gather(j)
            for j in range(NB2):
                wait_gather(j)
                scale_msg(j)
                fire_scatter(j)
            for j in range(NB2):
                cn = c0 + NB2 + j

                @pl.when(cn < NCH2)
                def _():
                    wait_scatter(j)
                    fire_stage(cn, j)
            return 0
        lax.fori_loop(0, NG2, group, 0)
        tbase = ebase + NCH2 * CH2
        wait_scatter(0)
        pltpu.sync_copy(rid_hbm.at[pl.ds(tbase, TAIL_E)],
                        row_b[0].at[pl.ds(0, TAIL_E)])
        pltpu.sync_copy(cid_hbm.at[pl.ds(tbase, TAIL_E)], col_t)
        pltpu.sync_copy(ew_hbm.at[pl.ds(tbase, TAIL_E)],
                        ew_b[0].at[pl.ds(0, TAIL_E)])
        row_b[0][pl.ds(0, 16)] = row_b[0][pl.ds(0, 16)] + coreofs
        pltpu.sync_copy(y_hbm.at[row_b[0].at[pl.ds(0, TAIL_E)]],
                        msg[0].at[pl.ds(0, TAIL_E)])
        def tmul_body(e, _):
            s = plsc.load_gather(ew_b[0], [jnp.full((16,), e, jnp.int32)])
            for v in range(HALF // 16):
                msg[0][e, pl.ds(v * 16, 16)] = msg[0][e, pl.ds(v * 16, 16)] * s
            return 0
        lax.fori_loop(0, TAIL_E, tmul_body, 0, unroll=8)
        pltpu.sync_copy(msg[0].at[pl.ds(0, TAIL_E)], acc.at[col_t], add=True)
        for j in range(1, NB2):
            wait_scatter(j)
        plsc.subcore_barrier()

        @pl.when(~last)
        def _():
            pltpu.sync_copy(acc.at[pl.ds(nbase, NT_BIG)],
                            out_hbm.at[pl.ds(coreofs + nbase, NT_BIG)])
        @pl.when(last)
        def _():
            pltpu.sync_copy(acc.at[pl.ds(nbase, NT_LAST)],
                            out_hbm.at[pl.ds(coreofs + nbase, NT_LAST)])

    return agg_kernel(y_flat, row_ids, col_ids, edge_weight)


BS = 1000


def _tc_matmul_kernel(x_ref, w_ref, dinv_ref, o_ref):
    res = lax.dot_general(x_ref[...], w_ref[...],
                          (((1,), (1,)), ((), ())),
                          preferred_element_type=jnp.float32)
    res = res * dinv_ref[...]
    o_ref[0] = res[:, :HALF]
    o_ref[1] = res[:, HALF:]


def _tc_head_kernel(a_ref, dinv_ref, sc0_ref, sc1_ref, sh0_ref, sh1_ref,
                    wih0a_ref, wih0b_ref, bi0_ref, bh0_ref,
                    wih1_ref, bi1_ref, bh1_ref,
                    w1_ref, b1_ref, w2_ref, b2_ref, o_ref):
    dv = dinv_ref[...]
    s0 = jnp.maximum((a_ref[0] * dv) * sc0_ref[...] + sh0_ref[...], 0.0)
    s1 = jnp.maximum((a_ref[1] * dv) * sc1_ref[...] + sh1_ref[...], 0.0)
    gi = (lax.dot_general(s0, wih0a_ref[...], (((1,), (1,)), ((), ())),
                          preferred_element_type=jnp.float32)
          + lax.dot_general(s1, wih0b_ref[...], (((1,), (1,)), ((), ())),
                            preferred_element_type=jnp.float32)
          + bi0_ref[...])
    bh0 = bh0_ref[...]
    H = D_GRU
    r = jax.nn.sigmoid(gi[:, :H] + bh0[:, :H])
    z = jax.nn.sigmoid(gi[:, H:2 * H] + bh0[:, H:2 * H])
    n_ = jnp.tanh(gi[:, 2 * H:] + r * bh0[:, 2 * H:])
    h1 = (1.0 - z) * n_
    gi2 = lax.dot_general(h1, wih1_ref[...], (((1,), (1,)), ((), ())),
                          preferred_element_type=jnp.float32) + bi1_ref[...]
    bh1 = bh1_ref[...]
    r2 = jax.nn.sigmoid(gi2[:, :H] + bh1[:, :H])
    z2 = jax.nn.sigmoid(gi2[:, H:2 * H] + bh1[:, H:2 * H])
    n2 = jnp.tanh(gi2[:, 2 * H:] + r2 * bh1[:, 2 * H:])
    h2 = (1.0 - z2) * n2
    hid = jnp.maximum(
        lax.dot_general(h2, w1_ref[...], (((1,), (1,)), ((), ())),
                        preferred_element_type=jnp.float32) + b1_ref[...], 0.0)
    o_ref[...] = (jnp.sum(hid * w2_ref[...], axis=1, keepdims=True)
                  + b2_ref[...])


def _full(shape):
    return pl.BlockSpec(shape, lambda i: tuple(0 for _ in shape))


def kernel(x, edge_index, edge_weight, W_gcn, b_gcn, bn_w, bn_b, bn_mean,
           bn_var, W_ih0, W_hh0, b_ih0, b_hh0, W_ih1, W_hh1, b_ih1, b_hh1,
           out_W1, out_b1, out_W2, out_b2):
    dinv_pad = _sc_degree(edge_index[1], edge_weight)
    dinv_col = dinv_pad[:N].reshape(N, 1)

    y2 = pl.pallas_call(
        _tc_matmul_kernel,
        grid=(N // BS,),
        in_specs=[pl.BlockSpec((BS, D_IN), lambda i: (i, 0)),
                  _full((D_GCN, D_IN)),
                  pl.BlockSpec((BS, 1), lambda i: (i, 0))],
        out_specs=pl.BlockSpec((2, BS, HALF), lambda i: (0, i, 0)),
        out_shape=jax.ShapeDtypeStruct((2, N, HALF), jnp.float32),
    )(x, W_gcn, dinv_col)
    y_flat = y2.reshape(2 * N, HALF)

    agg_flat = _sc_aggregate(y_flat, edge_index[0], edge_index[1],
                             edge_weight)
    agg = agg_flat.reshape(2, N, HALF)

    bscale = bn_w * lax.rsqrt(bn_var + 1e-5)
    bshift = (b_gcn - bn_mean) * bscale + bn_b
    score = pl.pallas_call(
        _tc_head_kernel,
        grid=(N // BS,),
        in_specs=[pl.BlockSpec((2, BS, HALF), lambda i: (0, i, 0)),
                  pl.BlockSpec((BS, 1), lambda i: (i, 0)),
                  _full((1, HALF)), _full((1, HALF)),
                  _full((1, HALF)), _full((1, HALF)),
                  _full((3 * D_GRU, HALF)), _full((3 * D_GRU, HALF)),
                  _full((1, 3 * D_GRU)), _full((1, 3 * D_GRU)),
                  _full((3 * D_GRU, D_GRU)),
                  _full((1, 3 * D_GRU)), _full((1, 3 * D_GRU)),
                  _full((D_GAT, D_GRU)), _full((1, D_GAT)),
                  _full((1, D_GAT)), _full((1, 1))],
        out_specs=pl.BlockSpec((BS, 1), lambda i: (i, 0)),
        out_shape=jax.ShapeDtypeStruct((N, 1), jnp.float32),
    )(agg, dinv_col,
      bscale[:HALF].reshape(1, HALF), bscale[HALF:].reshape(1, HALF),
      bshift[:HALF].reshape(1, HALF), bshift[HALF:].reshape(1, HALF),
      W_ih0[:, :HALF], W_ih0[:, HALF:],
      b_ih0.reshape(1, -1), b_hh0.reshape(1, -1),
      W_ih1, b_ih1.reshape(1, -1), b_hh1.reshape(1, -1),
      out_W1, out_b1.reshape(1, -1),
      out_W2, out_b2.reshape(1, -1))
    return score
